# baseline (speedup 1.0000x reference)
import jax
import jax.numpy as jnp
from jax.experimental import pallas as pl

N = 50000
B = 128
HID = 64
HEADS = 4
CH = HID // HEADS
NL = 4


def _layernorm(v, g, b):
    m = v.mean(-1, keepdims=True)
    var = v.var(-1, keepdims=True)
    return (v - m) / jnp.sqrt(var + 1e-5) * g + b


def _gat(h, src, dst, W, a_src, a_dst, bias):
    xh = (h @ W).reshape(h.shape[0], HEADS, CH)
    e = jax.nn.leaky_relu((xh * a_src).sum(-1)[src] + (xh * a_dst).sum(-1)[dst], 0.2)
    emax = jax.ops.segment_max(e, dst, num_segments=N)
    ee = jnp.exp(e - emax[dst])
    den = jax.ops.segment_sum(ee, dst, num_segments=N)
    w = ee / (den[dst] + 1e-16)
    out = jax.ops.segment_sum(xh[src] * w[:, :, None], dst, num_segments=N)
    return out.reshape(h.shape[0], HID) + bias


def _head_kernel(hc_ref, omw1_ref, omb1_ref, omw2_ref, omb2_ref,
                 cvw1_ref, cvb1_ref, cvw2_ref, cvb2_ref,
                 om_ref, cv_ref):
    hc = hc_ref[...]
    om_h = jnp.maximum(hc @ omw1_ref[...] + omb1_ref[...], 0.0)
    om = jax.nn.softplus(om_h @ omw2_ref[...] + omb2_ref[...])
    cv_h = jnp.maximum(hc @ cvw1_ref[...] + cvb1_ref[...], 0.0)
    cv = jax.nn.softplus(cv_h @ cvw2_ref[...] + cvb2_ref[...])
    om_ref[...] = om
    cv_ref[...] = cv


def kernel(x, edge_index, batch, global_features, params):
    p = params
    ar = jnp.arange(N, dtype=edge_index.dtype)
    src = jnp.concatenate([edge_index[0], ar])
    dst = jnp.concatenate([edge_index[1], ar])
    h = jax.nn.relu(x @ p['enc_W1'] + p['enc_b1']) @ p['enc_W2'] + p['enc_b2']
    for l in range(NL):
        g = _gat(h, src, dst, p[f'gat_W{l}'], p[f'att_src{l}'], p[f'att_dst{l}'], p[f'gat_b{l}'])
        h = jax.nn.relu(_layernorm(g + h, p[f'ln_g{l}'], p[f'ln_b{l}']))
    cnt = jax.ops.segment_sum(jnp.ones((N,), dtype=h.dtype), batch, num_segments=B)
    h_graph = jax.ops.segment_sum(h, batch, num_segments=B) / jnp.maximum(cnt, 1.0)[:, None]
    h_global = global_features @ p['glob_W'] + p['glob_b']
    hc = jnp.concatenate([h_graph, h_global], axis=-1)

    omega_raw, cv_direct = pl.pallas_call(
        _head_kernel,
        out_shape=(jax.ShapeDtypeStruct((B, 1), jnp.float32),
                   jax.ShapeDtypeStruct((B, 1), jnp.float32)),
    )(hc, p['om_W1'], p['om_b1'], p['om_W2'], p['om_b2'],
      p['cv_W1'], p['cv_b1'], p['cv_W2'], p['cv_b2'])

    alpha = jnp.exp(p['log_alpha'])
    omega_max = jnp.exp(p['log_omega_max'])
    omega_eff = jnp.minimum(alpha * omega_raw, omega_max)
    omega_eff = jnp.maximum(omega_eff, 2.0)
    lmbda = jnp.exp(p['log_lambda'])
    causality = omega_eff ** (-lmbda)
    cv_physics = 30.0 * (1.0 - causality)
    cv = 0.5 * cv_physics + 0.5 * cv_direct
    return (cv, cv_physics, cv_direct, omega_eff, omega_raw, causality, lmbda, alpha, omega_max)


# speedup vs baseline: 38.7663x; 38.7663x over previous
import functools

import jax
import jax.numpy as jnp
from jax import lax
from jax.experimental import pallas as pl
from jax.experimental.pallas import tpu as pltpu
from jax.experimental.pallas import tpu_sc as plsc

N = 50000
E = 800000
B = 128
AF = 21
HID = 64
HEADS = 4
CH = 16
NL = 4

BLK = 1024
NPAD = 50176
NB = NPAD // BLK
EACT = E + N
NSC = 2
NTI = 16
K = 256
NCHUNK = 208
TPT = K * NCHUNK
EP = TPT * NTI
RPT = NPAD // NTI
LK = 0.2


def _dense_pre(h, W, asrc, adst, S):
    xh = jnp.dot(h, W, preferred_element_type=jnp.float32)
    s = jnp.dot(xh * asrc, S, preferred_element_type=jnp.float32)
    d = jnp.dot(xh * adst, S, preferred_element_type=jnp.float32)
    return xh, s, d


def _acc_max(ref, blk, i):
    bm = jnp.max(blk, axis=0, keepdims=True)

    @pl.when(i == 0)
    def _():
        ref[...] = bm

    @pl.when(i > 0)
    def _():
        ref[...] = jnp.maximum(ref[...], bm)


def _enc_kernel(x_ref, w1_ref, b1_ref, w2_ref, b2_ref, W_ref, as_ref,
                ad_ref, S_ref, h_ref, xh_ref, s_ref, d_ref, sm_ref, dm_ref):
    i = pl.program_id(0)
    x = x_ref[...]
    t = jnp.maximum(jnp.dot(x, w1_ref[...], preferred_element_type=jnp.float32)
                    + b1_ref[...], 0.0)
    h = jnp.dot(t, w2_ref[...], preferred_element_type=jnp.float32) + b2_ref[...]
    h_ref[...] = h
    xh, s, d = _dense_pre(h, W_ref[...], as_ref[...], ad_ref[...], S_ref[...])
    xh_ref[...] = xh
    s_ref[...] = s
    d_ref[...] = d
    _acc_max(sm_ref, s, i)
    _acc_max(dm_ref, d, i)


def _post_body(o0_ref, o1_ref, dn0_ref, dn1_ref, hp_ref, bias_ref, lg_ref,
               lb_ref, S2_ref):
    d0 = jnp.dot(dn0_ref[0], S2_ref[...], preferred_element_type=jnp.float32)
    d1 = jnp.dot(dn1_ref[0], S2_ref[...], preferred_element_type=jnp.float32)
    g0 = o0_ref[0] / (d0 + 1e-16)
    g1 = o1_ref[0] / (d1 + 1e-16)
    g = jnp.concatenate([g0, g1], axis=1) + bias_ref[...]
    t = g + hp_ref[...]
    m = jnp.mean(t, axis=-1, keepdims=True)
    v = jnp.mean((t - m) * (t - m), axis=-1, keepdims=True)
    t = (t - m) / jnp.sqrt(v + 1e-5) * lg_ref[...] + lb_ref[...]
    return jnp.maximum(t, 0.0)


def _post_pre_kernel(o0_ref, o1_ref, dn0_ref, dn1_ref, hp_ref, bias_ref,
                     lg_ref, lb_ref, S2_ref, W_ref, as_ref, ad_ref, S_ref,
                     h_ref, xh_ref, s_ref, d_ref, sm_ref, dm_ref):
    i = pl.program_id(0)
    h = _post_body(o0_ref, o1_ref, dn0_ref, dn1_ref, hp_ref, bias_ref, lg_ref,
                   lb_ref, S2_ref)
    h_ref[...] = h
    xh, s, d = _dense_pre(h, W_ref[...], as_ref[...], ad_ref[...], S_ref[...])
    xh_ref[...] = xh
    s_ref[...] = s
    d_ref[...] = d
    _acc_max(sm_ref, s, i)
    _acc_max(dm_ref, d, i)


def _post_pool_kernel(o0_ref, o1_ref, dn0_ref, dn1_ref, hp_ref, bias_ref,
                      lg_ref, lb_ref, S2_ref, batch_ref, hsum_ref, cnt_ref):
    i = pl.program_id(0)
    h = _post_body(o0_ref, o1_ref, dn0_ref, dn1_ref, hp_ref, bias_ref, lg_ref,
                   lb_ref, S2_ref)
    iota_b = lax.broadcasted_iota(jnp.int32, (1, B), 1).astype(jnp.float32)
    onehot = (batch_ref[...] == iota_b).astype(jnp.float32)
    hs = lax.dot_general(onehot, h, (((0,), (0,)), ((), ())),
                         preferred_element_type=jnp.float32)
    cn = jnp.sum(onehot, axis=0, keepdims=True)

    @pl.when(i == 0)
    def _():
        hsum_ref[...] = hs
        cnt_ref[...] = cn

    @pl.when(i > 0)
    def _():
        hsum_ref[...] = hsum_ref[...] + hs
        cnt_ref[...] = cnt_ref[...] + cn


def _heads_kernel(hsum_ref, cnt_ref, gf_ref, gw_ref, gb_ref,
                  ow1_ref, ob1_ref, ow2_ref, ob2_ref,
                  cw1_ref, cb1_ref, cw2_ref, cb2_ref, logs_ref,
                  cv_ref, cvp_ref, cvd_ref, oe_ref, or_ref, ca_ref):
    cnt = jnp.maximum(cnt_ref[...], 1.0)
    h_graph = hsum_ref[...] / cnt
    h_glob = jnp.dot(gf_ref[...], gw_ref[...],
                     preferred_element_type=jnp.float32) + gb_ref[...]
    hc = jnp.concatenate([h_graph, h_glob], axis=1)
    th = jnp.maximum(jnp.dot(hc, ow1_ref[...],
                             preferred_element_type=jnp.float32) + ob1_ref[...], 0.0)
    omega_raw = jax.nn.softplus(
        jnp.dot(th, ow2_ref[...], preferred_element_type=jnp.float32) + ob2_ref[...])
    tc = jnp.maximum(jnp.dot(hc, cw1_ref[...],
                             preferred_element_type=jnp.float32) + cb1_ref[...], 0.0)
    cv_direct = jax.nn.softplus(
        jnp.dot(tc, cw2_ref[...], preferred_element_type=jnp.float32) + cb2_ref[...])
    lmbda = jnp.exp(logs_ref[0, 0])
    alpha = jnp.exp(logs_ref[0, 1])
    omega_max = jnp.exp(logs_ref[0, 2])
    omega_eff = jnp.maximum(jnp.minimum(alpha * omega_raw, omega_max), 2.0)
    causality = jnp.exp(-lmbda * jnp.log(omega_eff))
    cv_physics = 30.0 * (1.0 - causality)
    cv_ref[...] = 0.5 * cv_physics + 0.5 * cv_direct
    cvp_ref[...] = cv_physics
    cvd_ref[...] = cv_direct
    oe_ref[...] = omega_eff
    or_ref[...] = omega_raw
    ca_ref[...] = causality


def _sc_edge_kernel(src_hbm, dst_hbm, sflat_hbm, dflat_hbm, xh_hbm, m_hbm,
                    out_hbm, den_a, den_b, den_c, den_d,
                    srcb, dstb, sxi, s0i, s1i, d0i, d1i, dsti,
                    sv0, sv1, dv0, dv1, ee0, ee1, xhr, upd, m16v,
                    out_sh, den0_sh, den1_sh):
    c = lax.axis_index("c")
    t_id = lax.axis_index("s")
    rowbase = t_id * RPT
    tilebase = t_id * TPT

    def _zero(out_sh, den0_sh, den1_sh):
        zv = jnp.zeros((16,), jnp.float32)
        for r in range(K):
            upd[r, pl.ds(0, 16)] = zv
            upd[r, pl.ds(16, 16)] = zv
        for g in range(K // 16):
            ee0[pl.ds(g * 16, 16)] = zv
        for j in range(12):
            pltpu.sync_copy(upd, out_sh.at[pl.ds(rowbase + j * K, K), :])
            pltpu.sync_copy(ee0, den0_sh.at[pl.ds(rowbase + j * K, K)])
            pltpu.sync_copy(ee0, den1_sh.at[pl.ds(rowbase + j * K, K)])
        r = RPT - 12 * K
        pltpu.sync_copy(upd.at[pl.ds(0, r), :],
                        out_sh.at[pl.ds(rowbase + 12 * K, r), :])
        pltpu.sync_copy(ee0.at[pl.ds(0, r)],
                        den0_sh.at[pl.ds(rowbase + 12 * K, r)])
        pltpu.sync_copy(ee0.at[pl.ds(0, r)],
                        den1_sh.at[pl.ds(rowbase + 12 * K, r)])

    if True:
        pltpu.sync_copy(m_hbm, m16v)
        _zero(out_sh, den0_sh, den1_sh)
        plsc.subcore_barrier()

        mvec = m16v[...]
        iota16 = lax.broadcasted_iota(jnp.int32, (16,), 0)
        h0 = 2 * c
        msp0 = mvec.at[jnp.broadcast_to(h0, (16,))].get(mode="promise_in_bounds")
        msp1 = mvec.at[jnp.broadcast_to(h0 + 1, (16,))].get(mode="promise_in_bounds")
        offx = c * NPAD
        offs0 = h0 * NPAD
        offs1 = (h0 + 1) * NPAD

        def body(i, carry):
            ebase = pl.multiple_of(tilebase + i * K, K)
            pltpu.sync_copy(src_hbm.at[pl.ds(ebase, K)], srcb)
            pltpu.sync_copy(dst_hbm.at[pl.ds(ebase, K)], dstb)
            for g in range(K // 16):
                j, o = g // 8, (g % 8) * 16
                sv = srcb[pl.ds(g * 16, 16)]
                dv = dstb[pl.ds(g * 16, 16)]
                sxi[j, pl.ds(o, 16)] = sv + offx
                s0i[j, pl.ds(o, 16)] = sv + offs0
                s1i[j, pl.ds(o, 16)] = sv + offs1
                d0i[j, pl.ds(o, 16)] = dv + offs0
                d1i[j, pl.ds(o, 16)] = dv + offs1
                dsti[j, pl.ds(o, 16)] = dv
            for j in range(K // 128):
                pltpu.sync_copy(sflat_hbm.at[s0i.at[j]], sv0.at[pl.ds(j * 128, 128)])
                pltpu.sync_copy(sflat_hbm.at[s1i.at[j]], sv1.at[pl.ds(j * 128, 128)])
                pltpu.sync_copy(dflat_hbm.at[d0i.at[j]], dv0.at[pl.ds(j * 128, 128)])
                pltpu.sync_copy(dflat_hbm.at[d1i.at[j]], dv1.at[pl.ds(j * 128, 128)])
                pltpu.sync_copy(xh_hbm.at[sxi.at[j]],
                                xhr.at[pl.ds(j * 128, 128), :])
            for g in range(K // 16):
                pos = ebase + g * 16 + iota16
                valid = pos < EACT
                for (svr, dvr, msp, eer) in ((sv0, dv0, msp0, ee0),
                                             (sv1, dv1, msp1, ee1)):
                    v = svr[pl.ds(g * 16, 16)] + dvr[pl.ds(g * 16, 16)]
                    v = jnp.where(v >= 0.0, v, v * LK)
                    v = jnp.exp(v - msp)
                    eer[pl.ds(g * 16, 16)] = jnp.where(valid, v, 0.0)
            for g in range(K // 16):
                e0 = ee0[pl.ds(g * 16, 16)]
                e1 = ee1[pl.ds(g * 16, 16)]
                for r in range(16):
                    row = g * 16 + r
                    cidx = jnp.full((16,), r, jnp.int32)
                    sp0 = e0.at[cidx].get(mode="promise_in_bounds")
                    sp1 = e1.at[cidx].get(mode="promise_in_bounds")
                    upd[row, pl.ds(0, 16)] = xhr[row, pl.ds(0, 16)] * sp0
                    upd[row, pl.ds(16, 16)] = xhr[row, pl.ds(16, 16)] * sp1
            for j in range(K // 128):
                pltpu.sync_copy(ee0.at[pl.ds(j * 128, 128)],
                                den0_sh.at[dsti.at[j]], add=True)
                pltpu.sync_copy(ee1.at[pl.ds(j * 128, 128)],
                                den1_sh.at[dsti.at[j]], add=True)
                pltpu.sync_copy(upd.at[pl.ds(j * 128, 128), :],
                                out_sh.at[dsti.at[j]], add=True)
            return carry

        lax.fori_loop(0, NCHUNK, body, 0)
        plsc.subcore_barrier()
        pltpu.sync_copy(out_sh.at[pl.ds(rowbase, RPT), :],
                        out_hbm.at[c, pl.ds(rowbase, RPT), :])

        @pl.when(c == 0)
        def _():
            pltpu.sync_copy(den0_sh.at[pl.ds(rowbase, RPT)],
                            den_a.at[pl.ds(rowbase, RPT)])
            pltpu.sync_copy(den1_sh.at[pl.ds(rowbase, RPT)],
                            den_b.at[pl.ds(rowbase, RPT)])

        @pl.when(c == 1)
        def _():
            pltpu.sync_copy(den0_sh.at[pl.ds(rowbase, RPT)],
                            den_c.at[pl.ds(rowbase, RPT)])
            pltpu.sync_copy(den1_sh.at[pl.ds(rowbase, RPT)],
                            den_d.at[pl.ds(rowbase, RPT)])


_SC_EDGE = pl.kernel(
    _sc_edge_kernel,
    out_type=(jax.ShapeDtypeStruct((NSC, NPAD, 32), jnp.float32),
              jax.ShapeDtypeStruct((NPAD,), jnp.float32),
              jax.ShapeDtypeStruct((NPAD,), jnp.float32),
              jax.ShapeDtypeStruct((NPAD,), jnp.float32),
              jax.ShapeDtypeStruct((NPAD,), jnp.float32)),
    mesh=plsc.VectorSubcoreMesh(core_axis_name="c", subcore_axis_name="s"),
    compiler_params=pltpu.CompilerParams(use_tc_tiling_on_sc=False),
    scratch_types=[
        pltpu.VMEM((K,), jnp.int32),
        pltpu.VMEM((K,), jnp.int32),
        pltpu.VMEM((2, 128), jnp.int32),
        pltpu.VMEM((2, 128), jnp.int32),
        pltpu.VMEM((2, 128), jnp.int32),
        pltpu.VMEM((2, 128), jnp.int32),
        pltpu.VMEM((2, 128), jnp.int32),
        pltpu.VMEM((2, 128), jnp.int32),
        pltpu.VMEM((K,), jnp.float32),
        pltpu.VMEM((K,), jnp.float32),
        pltpu.VMEM((K,), jnp.float32),
        pltpu.VMEM((K,), jnp.float32),
        pltpu.VMEM((K,), jnp.float32),
        pltpu.VMEM((K,), jnp.float32),
        pltpu.VMEM((K, 32), jnp.float32),
        pltpu.VMEM((K, 32), jnp.float32),
        pltpu.VMEM((16,), jnp.float32),
        pltpu.VMEM_SHARED((NPAD, 32), jnp.float32),
        pltpu.VMEM_SHARED((NPAD,), jnp.float32),
        pltpu.VMEM_SHARED((NPAD,), jnp.float32),
    ],
)


def _tc_pre_specs():
    w = pl.BlockSpec((HID, HID), lambda i: (0, 0))
    v64 = pl.BlockSpec((1, HID), lambda i: (0, 0))
    s4 = pl.BlockSpec((HID, HEADS), lambda i: (0, 0))
    return w, v64, s4


def kernel(x, edge_index, batch, global_features, params):
    p = params
    f32 = jnp.float32

    ar = jnp.arange(N, dtype=jnp.int32)
    src = jnp.concatenate([edge_index[0].astype(jnp.int32), ar])
    dst = jnp.concatenate([edge_index[1].astype(jnp.int32), ar])
    src = jnp.pad(src, (0, EP - EACT))
    dst = jnp.pad(dst, (0, EP - EACT))

    x_p = jnp.pad(x, ((0, NPAD - N), (0, 0)))
    S = jnp.kron(jnp.eye(HEADS, dtype=f32), jnp.ones((CH, 1), f32))
    S2 = jnp.kron(jnp.eye(2, dtype=f32), jnp.ones((1, CH), f32))

    nblk = pl.BlockSpec((BLK, HID), lambda i: (i, 0))
    nblk4 = pl.BlockSpec((BLK, HEADS), lambda i: (i, 0))
    m4 = pl.BlockSpec((1, HEADS), lambda i: (0, 0))
    w, v64, s4 = _tc_pre_specs()

    h, xh, s, d, smax, dmax = pl.pallas_call(
        _enc_kernel,
        grid=(NB,),
        in_specs=[pl.BlockSpec((BLK, AF), lambda i: (i, 0)),
                  pl.BlockSpec((AF, HID), lambda i: (0, 0)), v64, w, v64,
                  w, v64, v64, s4],
        out_specs=[nblk, nblk, nblk4, nblk4, m4, m4],
        out_shape=[jax.ShapeDtypeStruct((NPAD, HID), f32),
                   jax.ShapeDtypeStruct((NPAD, HID), f32),
                   jax.ShapeDtypeStruct((NPAD, HEADS), f32),
                   jax.ShapeDtypeStruct((NPAD, HEADS), f32),
                   jax.ShapeDtypeStruct((1, HEADS), f32),
                   jax.ShapeDtypeStruct((1, HEADS), f32)],
    )(x_p, p['enc_W1'], p['enc_b1'].reshape(1, HID), p['enc_W2'],
      p['enc_b2'].reshape(1, HID), p['gat_W0'],
      p['att_src0'].reshape(1, HID), p['att_dst0'].reshape(1, HID), S)

    for l in range(NL):
        sflat = s.T.reshape(-1)
        dflat = d.T.reshape(-1)
        xh_cat = jnp.concatenate([xh[:, :32], xh[:, 32:]], axis=0)
        m16 = jnp.pad((smax + dmax).reshape(HEADS), (0, 12))
        out_sc, den_a, den_b, den_c, den_d = _SC_EDGE(
            src, dst, sflat, dflat, xh_cat, m16)
        den01 = jnp.stack([den_a, den_b], axis=-1).reshape(1, NPAD, 2)
        den23 = jnp.stack([den_c, den_d], axis=-1).reshape(1, NPAD, 2)

        common_in = [
            pl.BlockSpec((1, BLK, 32), lambda i: (0, i, 0)),
            pl.BlockSpec((1, BLK, 32), lambda i: (1, i, 0)),
            pl.BlockSpec((1, BLK, 2), lambda i: (0, i, 0)),
            pl.BlockSpec((1, BLK, 2), lambda i: (0, i, 0)),
            nblk, v64, v64, v64,
            pl.BlockSpec((2, 32), lambda i: (0, 0)),
        ]
        common_args = [out_sc, out_sc, den01, den23, h,
                       p[f'gat_b{l}'].reshape(1, HID),
                       p[f'ln_g{l}'].reshape(1, HID),
                       p[f'ln_b{l}'].reshape(1, HID), S2]
        if l < NL - 1:
            h, xh, s, d, smax, dmax = pl.pallas_call(
                _post_pre_kernel,
                grid=(NB,),
                in_specs=common_in + [w, v64, v64, s4],
                out_specs=[nblk, nblk, nblk4, nblk4, m4, m4],
                out_shape=[jax.ShapeDtypeStruct((NPAD, HID), f32),
                           jax.ShapeDtypeStruct((NPAD, HID), f32),
                           jax.ShapeDtypeStruct((NPAD, HEADS), f32),
                           jax.ShapeDtypeStruct((NPAD, HEADS), f32),
                           jax.ShapeDtypeStruct((1, HEADS), f32),
                           jax.ShapeDtypeStruct((1, HEADS), f32)],
            )(*common_args, p[f'gat_W{l + 1}'],
              p[f'att_src{l + 1}'].reshape(1, HID),
              p[f'att_dst{l + 1}'].reshape(1, HID), S)
        else:
            batch_p = jnp.pad(batch.astype(jnp.int32), (0, NPAD - N),
                              constant_values=B).astype(f32).reshape(NPAD, 1)
            hsum, cnt = pl.pallas_call(
                _post_pool_kernel,
                grid=(NB,),
                in_specs=common_in + [pl.BlockSpec((BLK, 1), lambda i: (i, 0))],
                out_specs=[pl.BlockSpec((B, HID), lambda i: (0, 0)),
                           pl.BlockSpec((1, B), lambda i: (0, 0))],
                out_shape=[jax.ShapeDtypeStruct((B, HID), f32),
                           jax.ShapeDtypeStruct((1, B), f32)],
            )(*common_args, batch_p)

    logs = jnp.stack([p['log_lambda'], p['log_alpha'],
                      p['log_omega_max']]).reshape(1, 3)
    cv, cv_physics, cv_direct, omega_eff, omega_raw, causality = pl.pallas_call(
        _heads_kernel,
        out_shape=[jax.ShapeDtypeStruct((B, 1), f32)] * 6,
    )(hsum, cnt.reshape(B, 1), global_features, p['glob_W'],
      p['glob_b'].reshape(1, HID), p['om_W1'], p['om_b1'].reshape(1, HID),
      p['om_W2'], p['om_b2'].reshape(1, 1), p['cv_W1'],
      p['cv_b1'].reshape(1, HID), p['cv_W2'], p['cv_b2'].reshape(1, 1), logs)

    lmbda = jnp.exp(p['log_lambda'])
    alpha = jnp.exp(p['log_alpha'])
    omega_max = jnp.exp(p['log_omega_max'])
    return (cv, cv_physics, cv_direct, omega_eff, omega_raw, causality,
            lmbda, alpha, omega_max)


# speedup vs baseline: 87.4361x; 2.2555x over previous
import functools

import jax
import jax.numpy as jnp
from jax import lax
from jax.experimental import pallas as pl
from jax.experimental.pallas import tpu as pltpu
from jax.experimental.pallas import tpu_sc as plsc

N = 50000
E = 800000
B = 128
AF = 21
HID = 64
HEADS = 4
CH = 16
NL = 4

BLK = 1024
NPAD = 50176
NB = NPAD // BLK
EACT = E + N
NSC = 2
NTI = 16
K = 128
NCHUNK = 416
TPT = K * NCHUNK
EP = TPT * NTI
RPT = NPAD // NTI
LK = 0.2
XW = 34


def _dense_pre(h, W, asrc, adst, S):
    xh = jnp.dot(h, W, preferred_element_type=jnp.float32)
    s = jnp.dot(xh * asrc, S, preferred_element_type=jnp.float32)
    d = jnp.dot(xh * adst, S, preferred_element_type=jnp.float32)
    return xh, s, d


def _acc_max(ref, blk, i):
    bm = jnp.max(blk, axis=0, keepdims=True)

    @pl.when(i == 0)
    def _():
        ref[...] = bm

    @pl.when(i > 0)
    def _():
        ref[...] = jnp.maximum(ref[...], bm)


def _enc_kernel(x_ref, w1_ref, b1_ref, w2_ref, b2_ref, W_ref, as_ref,
                ad_ref, S_ref, h_ref, xh_ref, s_ref, d_ref, sm_ref, dm_ref):
    i = pl.program_id(0)
    x = x_ref[...]
    t = jnp.maximum(jnp.dot(x, w1_ref[...], preferred_element_type=jnp.float32)
                    + b1_ref[...], 0.0)
    h = jnp.dot(t, w2_ref[...], preferred_element_type=jnp.float32) + b2_ref[...]
    h_ref[...] = h
    xh, s, d = _dense_pre(h, W_ref[...], as_ref[...], ad_ref[...], S_ref[...])
    xh_ref[...] = xh
    s_ref[...] = s
    d_ref[...] = d
    _acc_max(sm_ref, s, i)
    _acc_max(dm_ref, d, i)


def _post_body(o0_ref, o1_ref, dn0_ref, dn1_ref, hp_ref, bias_ref, lg_ref,
               lb_ref, S2_ref):
    d0 = jnp.dot(dn0_ref[0], S2_ref[...], preferred_element_type=jnp.float32)
    d1 = jnp.dot(dn1_ref[0], S2_ref[...], preferred_element_type=jnp.float32)
    g0 = o0_ref[0] / (d0 + 1e-16)
    g1 = o1_ref[0] / (d1 + 1e-16)
    g = jnp.concatenate([g0, g1], axis=1) + bias_ref[...]
    t = g + hp_ref[...]
    m = jnp.mean(t, axis=-1, keepdims=True)
    v = jnp.mean((t - m) * (t - m), axis=-1, keepdims=True)
    t = (t - m) / jnp.sqrt(v + 1e-5) * lg_ref[...] + lb_ref[...]
    return jnp.maximum(t, 0.0)


def _post_pre_kernel(o0_ref, o1_ref, dn0_ref, dn1_ref, hp_ref, bias_ref,
                     lg_ref, lb_ref, S2_ref, W_ref, as_ref, ad_ref, S_ref,
                     h_ref, xh_ref, s_ref, d_ref, sm_ref, dm_ref):
    i = pl.program_id(0)
    h = _post_body(o0_ref, o1_ref, dn0_ref, dn1_ref, hp_ref, bias_ref, lg_ref,
                   lb_ref, S2_ref)
    h_ref[...] = h
    xh, s, d = _dense_pre(h, W_ref[...], as_ref[...], ad_ref[...], S_ref[...])
    xh_ref[...] = xh
    s_ref[...] = s
    d_ref[...] = d
    _acc_max(sm_ref, s, i)
    _acc_max(dm_ref, d, i)


def _post_pool_kernel(o0_ref, o1_ref, dn0_ref, dn1_ref, hp_ref, bias_ref,
                      lg_ref, lb_ref, S2_ref, batch_ref, hsum_ref, cnt_ref):
    i = pl.program_id(0)
    h = _post_body(o0_ref, o1_ref, dn0_ref, dn1_ref, hp_ref, bias_ref, lg_ref,
                   lb_ref, S2_ref)
    iota_b = lax.broadcasted_iota(jnp.int32, (1, B), 1).astype(jnp.float32)
    onehot = (batch_ref[...] == iota_b).astype(jnp.float32)
    hs = lax.dot_general(onehot, h, (((0,), (0,)), ((), ())),
                         preferred_element_type=jnp.float32)
    cn = jnp.sum(onehot, axis=0, keepdims=True)

    @pl.when(i == 0)
    def _():
        hsum_ref[...] = hs
        cnt_ref[...] = cn

    @pl.when(i > 0)
    def _():
        hsum_ref[...] = hsum_ref[...] + hs
        cnt_ref[...] = cnt_ref[...] + cn


def _heads_kernel(hsum_ref, cnt_ref, gf_ref, gw_ref, gb_ref,
                  ow1_ref, ob1_ref, ow2_ref, ob2_ref,
                  cw1_ref, cb1_ref, cw2_ref, cb2_ref, logs_ref,
                  cv_ref, cvp_ref, cvd_ref, oe_ref, or_ref, ca_ref):
    cnt = jnp.maximum(cnt_ref[...], 1.0)
    h_graph = hsum_ref[...] / cnt
    h_glob = jnp.dot(gf_ref[...], gw_ref[...],
                     preferred_element_type=jnp.float32) + gb_ref[...]
    hc = jnp.concatenate([h_graph, h_glob], axis=1)
    th = jnp.maximum(jnp.dot(hc, ow1_ref[...],
                             preferred_element_type=jnp.float32) + ob1_ref[...], 0.0)
    omega_raw = jax.nn.softplus(
        jnp.dot(th, ow2_ref[...], preferred_element_type=jnp.float32) + ob2_ref[...])
    tc = jnp.maximum(jnp.dot(hc, cw1_ref[...],
                             preferred_element_type=jnp.float32) + cb1_ref[...], 0.0)
    cv_direct = jax.nn.softplus(
        jnp.dot(tc, cw2_ref[...], preferred_element_type=jnp.float32) + cb2_ref[...])
    lmbda = jnp.exp(logs_ref[0, 0])
    alpha = jnp.exp(logs_ref[0, 1])
    omega_max = jnp.exp(logs_ref[0, 2])
    omega_eff = jnp.maximum(jnp.minimum(alpha * omega_raw, omega_max), 2.0)
    causality = jnp.exp(-lmbda * jnp.log(omega_eff))
    cv_physics = 30.0 * (1.0 - causality)
    cv_ref[...] = 0.5 * cv_physics + 0.5 * cv_direct
    cvp_ref[...] = cv_physics
    cvd_ref[...] = cv_direct
    oe_ref[...] = omega_eff
    or_ref[...] = omega_raw
    ca_ref[...] = causality


def _sc_edge_kernel(src_hbm, dst_hbm, sflat_hbm, dflat_hbm, xh_hbm, m_hbm,
                    out_hbm, den_a, den_b, den_c, den_d,
                    srcb0, dstb0, sxi0, s0i0, s1i0, d0i0, d1i0, dsti0,
                    sv00, sv10, dv00, dv10, xhr0, ee00, ee10, upd0,
                    srcb1, dstb1, sxi1, s0i1, s1i1, d0i1, d1i1, dsti1,
                    sv01, sv11, dv01, dv11, xhr1, ee01, ee11, upd1,
                    m16v, gsem0, gsem1, ssem, out_sh, den0_sh, den1_sh):
    c = lax.axis_index("c")
    t_id = lax.axis_index("s")
    rowbase = t_id * RPT
    tilebase = t_id * TPT
    P0 = (srcb0, dstb0, sxi0, s0i0, s1i0, d0i0, d1i0, dsti0,
          sv00, sv10, dv00, dv10, xhr0, ee00, ee10, upd0)
    P1 = (srcb1, dstb1, sxi1, s0i1, s1i1, d0i1, d1i1, dsti1,
          sv01, sv11, dv01, dv11, xhr1, ee01, ee11, upd1)

    def _zero(out_sh, den0_sh, den1_sh):
        zv = jnp.zeros((16,), jnp.float32)
        for r in range(K):
            upd0[r, pl.ds(0, 16)] = zv
            upd0[r, pl.ds(16, 16)] = zv
        for g in range(K // 16):
            ee00[pl.ds(g * 16, 16)] = zv
        nfull = RPT // K
        for j in range(nfull):
            pltpu.sync_copy(upd0, out_sh.at[pl.ds(rowbase + j * K, K), :])
            pltpu.sync_copy(ee00, den0_sh.at[pl.ds(rowbase + j * K, K)])
            pltpu.sync_copy(ee00, den1_sh.at[pl.ds(rowbase + j * K, K)])
        r = RPT - nfull * K
        pltpu.sync_copy(upd0.at[pl.ds(0, r), :],
                        out_sh.at[pl.ds(rowbase + nfull * K, r), :])
        pltpu.sync_copy(ee00.at[pl.ds(0, r)],
                        den0_sh.at[pl.ds(rowbase + nfull * K, r)])
        pltpu.sync_copy(ee00.at[pl.ds(0, r)],
                        den1_sh.at[pl.ds(rowbase + nfull * K, r)])

    if True:
        pltpu.sync_copy(m_hbm, m16v)
        _zero(out_sh, den0_sh, den1_sh)
        plsc.subcore_barrier()

        mvec = m16v[...]
        iota16 = lax.broadcasted_iota(jnp.int32, (16,), 0)
        h0 = 2 * c
        msp0 = mvec.at[jnp.broadcast_to(h0, (16,))].get(mode="promise_in_bounds")
        msp1 = mvec.at[jnp.broadcast_to(h0 + 1, (16,))].get(mode="promise_in_bounds")
        offx = c * NPAD
        offs0 = h0 * NPAD
        offs1 = (h0 + 1) * NPAD

        def _prep(P, ebase, gsem):
            srcb, dstb, sxi, s0i, s1i, d0i, d1i, dsti = P[:8]
            pltpu.sync_copy(src_hbm.at[pl.ds(ebase, K)], srcb)
            pltpu.sync_copy(dst_hbm.at[pl.ds(ebase, K)], dstb)
            for g in range(K // 16):
                o = g * 16
                sv = srcb[pl.ds(o, 16)]
                dv = dstb[pl.ds(o, 16)]
                sxi[pl.ds(o, 16)] = sv + offx
                s0i[pl.ds(o, 16)] = sv + offs0
                s1i[pl.ds(o, 16)] = sv + offs1
                d0i[pl.ds(o, 16)] = dv + offs0
                d1i[pl.ds(o, 16)] = dv + offs1
                dsti[pl.ds(o, 16)] = dv
            pltpu.async_copy(xh_hbm.at[P[2]], P[12], gsem)
            pltpu.async_copy(sflat_hbm.at[P[3]], P[8], gsem)
            pltpu.async_copy(sflat_hbm.at[P[4]], P[9], gsem)
            pltpu.async_copy(dflat_hbm.at[P[5]], P[10], gsem)
            pltpu.async_copy(dflat_hbm.at[P[6]], P[11], gsem)

        def _wait_gathers(P, gsem):
            pltpu.make_async_copy(xh_hbm.at[P[2]], P[12], gsem).wait()
            pltpu.make_async_copy(sflat_hbm.at[P[3]], P[8], gsem).wait()
            pltpu.make_async_copy(sflat_hbm.at[P[4]], P[9], gsem).wait()
            pltpu.make_async_copy(dflat_hbm.at[P[5]], P[10], gsem).wait()
            pltpu.make_async_copy(dflat_hbm.at[P[6]], P[11], gsem).wait()

        def _compute(P, ebase):
            sv0, sv1, dv0, dv1, xhr, ee0, ee1, upd = P[8:]
            for g in range(K // 16):
                o = g * 16
                pos = ebase + o + iota16
                valid = pos < EACT
                for (svr, dvr, msp, eer) in ((sv0, dv0, msp0, ee0),
                                             (sv1, dv1, msp1, ee1)):
                    v = svr[pl.ds(o, 16)] + dvr[pl.ds(o, 16)]
                    v = jnp.where(v >= 0.0, v, v * LK)
                    v = jnp.exp(v - msp)
                    eer[pl.ds(o, 16)] = jnp.where(valid, v, 0.0)
            for g in range(K // 16):
                e0 = ee0[pl.ds(g * 16, 16)]
                e1 = ee1[pl.ds(g * 16, 16)]
                for r in range(16):
                    row = g * 16 + r
                    cidx = jnp.full((16,), r, jnp.int32)
                    sp0 = e0.at[cidx].get(mode="promise_in_bounds")
                    sp1 = e1.at[cidx].get(mode="promise_in_bounds")
                    upd[row, pl.ds(0, 16)] = xhr[row, pl.ds(0, 16)] * sp0
                    upd[row, pl.ds(16, 16)] = xhr[row, pl.ds(16, 16)] * sp1

        def _fire_scatters(P):
            pltpu.async_copy(P[13], den0_sh.at[P[7]], ssem, add=True)
            pltpu.async_copy(P[14], den1_sh.at[P[7]], ssem, add=True)
            pltpu.async_copy(P[15], out_sh.at[P[7]], ssem, add=True)

        def _drain_scatters(P):
            pltpu.make_async_copy(P[13], den0_sh.at[P[7]], ssem).wait()
            pltpu.make_async_copy(P[14], den1_sh.at[P[7]], ssem).wait()
            pltpu.make_async_copy(P[15], out_sh.at[P[7]], ssem).wait()

        def body(i2, carry):
            ea = pl.multiple_of(tilebase + i2 * (2 * K), K)
            eb = pl.multiple_of(ea + K, K)

            @pl.when(i2 > 0)
            def _():
                _drain_scatters(P0)
                _drain_scatters(P1)

            _prep(P0, ea, gsem0)
            _prep(P1, eb, gsem1)
            _wait_gathers(P0, gsem0)
            _compute(P0, ea)
            _fire_scatters(P0)
            _wait_gathers(P1, gsem1)
            _compute(P1, eb)
            _fire_scatters(P1)
            return carry

        lax.fori_loop(0, NCHUNK // 2, body, 0)
        _drain_scatters(P0)
        _drain_scatters(P1)
        plsc.subcore_barrier()
        pltpu.sync_copy(out_sh.at[pl.ds(rowbase, RPT), :],
                        out_hbm.at[c, pl.ds(rowbase, RPT), :])

        @pl.when(c == 0)
        def _():
            pltpu.sync_copy(den0_sh.at[pl.ds(rowbase, RPT)],
                            den_a.at[pl.ds(rowbase, RPT)])
            pltpu.sync_copy(den1_sh.at[pl.ds(rowbase, RPT)],
                            den_b.at[pl.ds(rowbase, RPT)])

        @pl.when(c == 1)
        def _():
            pltpu.sync_copy(den0_sh.at[pl.ds(rowbase, RPT)],
                            den_c.at[pl.ds(rowbase, RPT)])
            pltpu.sync_copy(den1_sh.at[pl.ds(rowbase, RPT)],
                            den_d.at[pl.ds(rowbase, RPT)])


_SC_EDGE = pl.kernel(
    _sc_edge_kernel,
    out_type=(jax.ShapeDtypeStruct((NSC, NPAD, 32), jnp.float32),
              jax.ShapeDtypeStruct((NPAD,), jnp.float32),
              jax.ShapeDtypeStruct((NPAD,), jnp.float32),
              jax.ShapeDtypeStruct((NPAD,), jnp.float32),
              jax.ShapeDtypeStruct((NPAD,), jnp.float32)),
    mesh=plsc.VectorSubcoreMesh(core_axis_name="c", subcore_axis_name="s"),
    compiler_params=pltpu.CompilerParams(use_tc_tiling_on_sc=False),
    scratch_types=(
        [pltpu.VMEM((K,), jnp.int32),
         pltpu.VMEM((K,), jnp.int32),
         pltpu.VMEM((K,), jnp.int32),
         pltpu.VMEM((K,), jnp.int32),
         pltpu.VMEM((K,), jnp.int32),
         pltpu.VMEM((K,), jnp.int32),
         pltpu.VMEM((K,), jnp.int32),
         pltpu.VMEM((K,), jnp.int32),
         pltpu.VMEM((K,), jnp.float32),
         pltpu.VMEM((K,), jnp.float32),
         pltpu.VMEM((K,), jnp.float32),
         pltpu.VMEM((K,), jnp.float32),
         pltpu.VMEM((K, 32), jnp.float32),
         pltpu.VMEM((K,), jnp.float32),
         pltpu.VMEM((K,), jnp.float32),
         pltpu.VMEM((K, 32), jnp.float32),
         ] * 2
        + [pltpu.VMEM((16,), jnp.float32),
           pltpu.SemaphoreType.DMA,
           pltpu.SemaphoreType.DMA,
           pltpu.SemaphoreType.DMA,
           pltpu.VMEM_SHARED((NPAD, 32), jnp.float32),
           pltpu.VMEM_SHARED((NPAD,), jnp.float32),
           pltpu.VMEM_SHARED((NPAD,), jnp.float32),
           ]),
)


def _tc_pre_specs():
    w = pl.BlockSpec((HID, HID), lambda i: (0, 0))
    v64 = pl.BlockSpec((1, HID), lambda i: (0, 0))
    s4 = pl.BlockSpec((HID, HEADS), lambda i: (0, 0))
    return w, v64, s4


def kernel(x, edge_index, batch, global_features, params):
    p = params
    f32 = jnp.float32

    ar = jnp.arange(N, dtype=jnp.int32)
    src = jnp.concatenate([edge_index[0].astype(jnp.int32), ar])
    dst = jnp.concatenate([edge_index[1].astype(jnp.int32), ar])
    src = jnp.pad(src, (0, EP - EACT))
    dst = jnp.pad(dst, (0, EP - EACT))

    x_p = jnp.pad(x, ((0, NPAD - N), (0, 0)))
    S = jnp.kron(jnp.eye(HEADS, dtype=f32), jnp.ones((CH, 1), f32))
    S2 = jnp.kron(jnp.eye(2, dtype=f32), jnp.ones((1, CH), f32))

    nblk = pl.BlockSpec((BLK, HID), lambda i: (i, 0))
    nblk4 = pl.BlockSpec((BLK, HEADS), lambda i: (i, 0))
    m4 = pl.BlockSpec((1, HEADS), lambda i: (0, 0))
    w, v64, s4 = _tc_pre_specs()

    h, xh, s, d, smax, dmax = pl.pallas_call(
        _enc_kernel,
        grid=(NB,),
        in_specs=[pl.BlockSpec((BLK, AF), lambda i: (i, 0)),
                  pl.BlockSpec((AF, HID), lambda i: (0, 0)), v64, w, v64,
                  w, v64, v64, s4],
        out_specs=[nblk, nblk, nblk4, nblk4, m4, m4],
        out_shape=[jax.ShapeDtypeStruct((NPAD, HID), f32),
                   jax.ShapeDtypeStruct((NPAD, HID), f32),
                   jax.ShapeDtypeStruct((NPAD, HEADS), f32),
                   jax.ShapeDtypeStruct((NPAD, HEADS), f32),
                   jax.ShapeDtypeStruct((1, HEADS), f32),
                   jax.ShapeDtypeStruct((1, HEADS), f32)],
    )(x_p, p['enc_W1'], p['enc_b1'].reshape(1, HID), p['enc_W2'],
      p['enc_b2'].reshape(1, HID), p['gat_W0'],
      p['att_src0'].reshape(1, HID), p['att_dst0'].reshape(1, HID), S)

    for l in range(NL):
        sflat = s.T.reshape(-1)
        dflat = d.T.reshape(-1)
        xh_cat = jnp.concatenate([xh[:, :32], xh[:, 32:]], axis=0)
        m16 = jnp.pad((smax + dmax).reshape(HEADS), (0, 12))
        out_sc, den_a, den_b, den_c, den_d = _SC_EDGE(
            src, dst, sflat, dflat, xh_cat, m16)
        den01 = jnp.stack([den_a, den_b], axis=-1).reshape(1, NPAD, 2)
        den23 = jnp.stack([den_c, den_d], axis=-1).reshape(1, NPAD, 2)

        common_in = [
            pl.BlockSpec((1, BLK, 32), lambda i: (0, i, 0)),
            pl.BlockSpec((1, BLK, 32), lambda i: (1, i, 0)),
            pl.BlockSpec((1, BLK, 2), lambda i: (0, i, 0)),
            pl.BlockSpec((1, BLK, 2), lambda i: (0, i, 0)),
            nblk, v64, v64, v64,
            pl.BlockSpec((2, 32), lambda i: (0, 0)),
        ]
        common_args = [out_sc, out_sc, den01, den23, h,
                       p[f'gat_b{l}'].reshape(1, HID),
                       p[f'ln_g{l}'].reshape(1, HID),
                       p[f'ln_b{l}'].reshape(1, HID), S2]
        if l < NL - 1:
            h, xh, s, d, smax, dmax = pl.pallas_call(
                _post_pre_kernel,
                grid=(NB,),
                in_specs=common_in + [w, v64, v64, s4],
                out_specs=[nblk, nblk, nblk4, nblk4, m4, m4],
                out_shape=[jax.ShapeDtypeStruct((NPAD, HID), f32),
                           jax.ShapeDtypeStruct((NPAD, HID), f32),
                           jax.ShapeDtypeStruct((NPAD, HEADS), f32),
                           jax.ShapeDtypeStruct((NPAD, HEADS), f32),
                           jax.ShapeDtypeStruct((1, HEADS), f32),
                           jax.ShapeDtypeStruct((1, HEADS), f32)],
            )(*common_args, p[f'gat_W{l + 1}'],
              p[f'att_src{l + 1}'].reshape(1, HID),
              p[f'att_dst{l + 1}'].reshape(1, HID), S)
        else:
            batch_p = jnp.pad(batch.astype(jnp.int32), (0, NPAD - N),
                              constant_values=B).astype(f32).reshape(NPAD, 1)
            hsum, cnt = pl.pallas_call(
                _post_pool_kernel,
                grid=(NB,),
                in_specs=common_in + [pl.BlockSpec((BLK, 1), lambda i: (i, 0))],
                out_specs=[pl.BlockSpec((B, HID), lambda i: (0, 0)),
                           pl.BlockSpec((1, B), lambda i: (0, 0))],
                out_shape=[jax.ShapeDtypeStruct((B, HID), f32),
                           jax.ShapeDtypeStruct((1, B), f32)],
            )(*common_args, batch_p)

    logs = jnp.stack([p['log_lambda'], p['log_alpha'],
                      p['log_omega_max']]).reshape(1, 3)
    cv, cv_physics, cv_direct, omega_eff, omega_raw, causality = pl.pallas_call(
        _heads_kernel,
        out_shape=[jax.ShapeDtypeStruct((B, 1), f32)] * 6,
    )(hsum, cnt.reshape(B, 1), global_features, p['glob_W'],
      p['glob_b'].reshape(1, HID), p['om_W1'], p['om_b1'].reshape(1, HID),
      p['om_W2'], p['om_b2'].reshape(1, 1), p['cv_W1'],
      p['cv_b1'].reshape(1, HID), p['cv_W2'], p['cv_b2'].reshape(1, 1), logs)

    lmbda = jnp.exp(p['log_lambda'])
    alpha = jnp.exp(p['log_alpha'])
    omega_max = jnp.exp(p['log_omega_max'])
    return (cv, cv_physics, cv_direct, omega_eff, omega_raw, causality,
            lmbda, alpha, omega_max)


# speedup vs baseline: 112.1303x; 1.2824x over previous
import functools

import jax
import jax.numpy as jnp
from jax import lax
from jax.experimental import pallas as pl
from jax.experimental.pallas import tpu as pltpu
from jax.experimental.pallas import tpu_sc as plsc

N = 50000
E = 800000
B = 128
AF = 21
HID = 64
HEADS = 4
CH = 16
NL = 4

BLK = 1024
NPAD = 50176
NB = NPAD // BLK
EACT = E + N
NSC = 2
NTI = 16
K = 128
NCHUNK = 416
TPT = K * NCHUNK
EP = TPT * NTI
RPT = NPAD // NTI
LK = 0.2
XW = 34


def _dense_pre(h, W, asrc, adst, S):
    xh = jnp.dot(h, W, preferred_element_type=jnp.float32)
    s = jnp.dot(xh * asrc, S, preferred_element_type=jnp.float32)
    d = jnp.dot(xh * adst, S, preferred_element_type=jnp.float32)
    return xh, s, d


def _acc_max(ref, blk, i):
    bm = jnp.max(blk, axis=0, keepdims=True)

    @pl.when(i == 0)
    def _():
        ref[...] = bm

    @pl.when(i > 0)
    def _():
        ref[...] = jnp.maximum(ref[...], bm)


def _enc_kernel(x_ref, w1_ref, b1_ref, w2_ref, b2_ref, W_ref, as_ref,
                ad_ref, S_ref, h_ref, xh_ref, s_ref, d_ref, sm_ref, dm_ref):
    i = pl.program_id(0)
    x = x_ref[...]
    t = jnp.maximum(jnp.dot(x, w1_ref[...], preferred_element_type=jnp.float32)
                    + b1_ref[...], 0.0)
    h = jnp.dot(t, w2_ref[...], preferred_element_type=jnp.float32) + b2_ref[...]
    h_ref[...] = h
    xh, s, d = _dense_pre(h, W_ref[...], as_ref[...], ad_ref[...], S_ref[...])
    xh_ref[...] = xh
    s_ref[...] = s
    d_ref[...] = d
    _acc_max(sm_ref, s, i)
    _acc_max(dm_ref, d, i)


def _post_body(o0_ref, o1_ref, dn0_ref, dn1_ref, hp_ref, bias_ref, lg_ref,
               lb_ref, S2_ref):
    d0 = jnp.dot(dn0_ref[0], S2_ref[...], preferred_element_type=jnp.float32)
    d1 = jnp.dot(dn1_ref[0], S2_ref[...], preferred_element_type=jnp.float32)
    g0 = o0_ref[0] / (d0 + 1e-16)
    g1 = o1_ref[0] / (d1 + 1e-16)
    g = jnp.concatenate([g0, g1], axis=1) + bias_ref[...]
    t = g + hp_ref[...]
    m = jnp.mean(t, axis=-1, keepdims=True)
    v = jnp.mean((t - m) * (t - m), axis=-1, keepdims=True)
    t = (t - m) / jnp.sqrt(v + 1e-5) * lg_ref[...] + lb_ref[...]
    return jnp.maximum(t, 0.0)


def _post_pre_kernel(o0_ref, o1_ref, dn0_ref, dn1_ref, hp_ref, bias_ref,
                     lg_ref, lb_ref, S2_ref, W_ref, as_ref, ad_ref, S_ref,
                     h_ref, xh_ref, s_ref, d_ref, sm_ref, dm_ref):
    i = pl.program_id(0)
    h = _post_body(o0_ref, o1_ref, dn0_ref, dn1_ref, hp_ref, bias_ref, lg_ref,
                   lb_ref, S2_ref)
    h_ref[...] = h
    xh, s, d = _dense_pre(h, W_ref[...], as_ref[...], ad_ref[...], S_ref[...])
    xh_ref[...] = xh
    s_ref[...] = s
    d_ref[...] = d
    _acc_max(sm_ref, s, i)
    _acc_max(dm_ref, d, i)


def _post_pool_kernel(o0_ref, o1_ref, dn0_ref, dn1_ref, hp_ref, bias_ref,
                      lg_ref, lb_ref, S2_ref, batch_ref, hsum_ref, cnt_ref):
    i = pl.program_id(0)
    h = _post_body(o0_ref, o1_ref, dn0_ref, dn1_ref, hp_ref, bias_ref, lg_ref,
                   lb_ref, S2_ref)
    iota_b = lax.broadcasted_iota(jnp.int32, (1, B), 1).astype(jnp.float32)
    onehot = (batch_ref[...] == iota_b).astype(jnp.float32)
    hs = lax.dot_general(onehot, h, (((0,), (0,)), ((), ())),
                         preferred_element_type=jnp.float32)
    cn = jnp.sum(onehot, axis=0, keepdims=True)

    @pl.when(i == 0)
    def _():
        hsum_ref[...] = hs
        cnt_ref[...] = cn

    @pl.when(i > 0)
    def _():
        hsum_ref[...] = hsum_ref[...] + hs
        cnt_ref[...] = cnt_ref[...] + cn


def _heads_kernel(hsum_ref, cnt_ref, gf_ref, gw_ref, gb_ref,
                  ow1_ref, ob1_ref, ow2_ref, ob2_ref,
                  cw1_ref, cb1_ref, cw2_ref, cb2_ref, logs_ref,
                  cv_ref, cvp_ref, cvd_ref, oe_ref, or_ref, ca_ref):
    cnt = jnp.maximum(cnt_ref[...], 1.0)
    h_graph = hsum_ref[...] / cnt
    h_glob = jnp.dot(gf_ref[...], gw_ref[...],
                     preferred_element_type=jnp.float32) + gb_ref[...]
    hc = jnp.concatenate([h_graph, h_glob], axis=1)
    th = jnp.maximum(jnp.dot(hc, ow1_ref[...],
                             preferred_element_type=jnp.float32) + ob1_ref[...], 0.0)
    omega_raw = jax.nn.softplus(
        jnp.dot(th, ow2_ref[...], preferred_element_type=jnp.float32) + ob2_ref[...])
    tc = jnp.maximum(jnp.dot(hc, cw1_ref[...],
                             preferred_element_type=jnp.float32) + cb1_ref[...], 0.0)
    cv_direct = jax.nn.softplus(
        jnp.dot(tc, cw2_ref[...], preferred_element_type=jnp.float32) + cb2_ref[...])
    lmbda = jnp.exp(logs_ref[0, 0])
    alpha = jnp.exp(logs_ref[0, 1])
    omega_max = jnp.exp(logs_ref[0, 2])
    omega_eff = jnp.maximum(jnp.minimum(alpha * omega_raw, omega_max), 2.0)
    causality = jnp.exp(-lmbda * jnp.log(omega_eff))
    cv_physics = 30.0 * (1.0 - causality)
    cv_ref[...] = 0.5 * cv_physics + 0.5 * cv_direct
    cvp_ref[...] = cv_physics
    cvd_ref[...] = cv_direct
    oe_ref[...] = omega_eff
    or_ref[...] = omega_raw
    ca_ref[...] = causality


def _sc_edge_kernel(src_hbm, dst_hbm, sflat_hbm, dflat_hbm, xh_hbm, m_hbm,
                    out_hbm, den_a, den_b, den_c, den_d,
                    sxi0, s0i0, s1i0, d0i0, d1i0, dsti0,
                    sv00, sv10, dv00, dv10, xhr0, ee00, ee10, upd0,
                    sxi1, s0i1, s1i1, d0i1, d1i1, dsti1,
                    sv01, sv11, dv01, dv11, xhr1, ee01, ee11, upd1,
                    lsrc, ldst, m16v, gsem0, gsem1, ssem, lsem,
                    out_sh, den0_sh, den1_sh):
    c = lax.axis_index("c")
    t_id = lax.axis_index("s")
    rowbase = t_id * RPT
    tilebase = t_id * TPT
    P0 = (0, sxi0, s0i0, s1i0, d0i0, d1i0, dsti0,
          sv00, sv10, dv00, dv10, xhr0, ee00, ee10, upd0)
    P1 = (K, sxi1, s0i1, s1i1, d0i1, d1i1, dsti1,
          sv01, sv11, dv01, dv11, xhr1, ee01, ee11, upd1)

    def _zero(out_sh, den0_sh, den1_sh):
        zv = jnp.zeros((16,), jnp.float32)
        for r in range(K):
            upd0[r, pl.ds(0, 16)] = zv
            upd0[r, pl.ds(16, 16)] = zv
        for g in range(K // 16):
            ee00[pl.ds(g * 16, 16)] = zv
        nfull = RPT // K
        for j in range(nfull):
            pltpu.sync_copy(upd0, out_sh.at[pl.ds(rowbase + j * K, K), :])
            pltpu.sync_copy(ee00, den0_sh.at[pl.ds(rowbase + j * K, K)])
            pltpu.sync_copy(ee00, den1_sh.at[pl.ds(rowbase + j * K, K)])
        r = RPT - nfull * K
        pltpu.sync_copy(upd0.at[pl.ds(0, r), :],
                        out_sh.at[pl.ds(rowbase + nfull * K, r), :])
        pltpu.sync_copy(ee00.at[pl.ds(0, r)],
                        den0_sh.at[pl.ds(rowbase + nfull * K, r)])
        pltpu.sync_copy(ee00.at[pl.ds(0, r)],
                        den1_sh.at[pl.ds(rowbase + nfull * K, r)])

    if True:
        pltpu.sync_copy(m_hbm, m16v)
        _zero(out_sh, den0_sh, den1_sh)
        plsc.subcore_barrier()

        mvec = m16v[...]
        iota16 = lax.broadcasted_iota(jnp.int32, (16,), 0)
        h0 = 2 * c
        msp0 = mvec.at[jnp.broadcast_to(h0, (16,))].get(mode="promise_in_bounds")
        msp1 = mvec.at[jnp.broadcast_to(h0 + 1, (16,))].get(mode="promise_in_bounds")
        offx = c * NPAD
        offs0 = h0 * NPAD
        offs1 = (h0 + 1) * NPAD

        def _prep(P, gsem):
            off, sxi, s0i, s1i, d0i, d1i, dsti = P[:7]
            for g in range(K // 16):
                o = g * 16
                sv = lsrc[pl.ds(off + o, 16)]
                dv = ldst[pl.ds(off + o, 16)]
                sxi[pl.ds(o, 16)] = sv + offx
                s0i[pl.ds(o, 16)] = sv + offs0
                s1i[pl.ds(o, 16)] = sv + offs1
                d0i[pl.ds(o, 16)] = dv + offs0
                d1i[pl.ds(o, 16)] = dv + offs1
                dsti[pl.ds(o, 16)] = dv
            pltpu.async_copy(xh_hbm.at[P[1]], P[11], gsem)
            pltpu.async_copy(sflat_hbm.at[P[2]], P[7], gsem)
            pltpu.async_copy(sflat_hbm.at[P[3]], P[8], gsem)
            pltpu.async_copy(dflat_hbm.at[P[4]], P[9], gsem)
            pltpu.async_copy(dflat_hbm.at[P[5]], P[10], gsem)

        def _wait_gathers(P, gsem):
            pltpu.make_async_copy(xh_hbm.at[P[1]], P[11], gsem).wait()
            pltpu.make_async_copy(sflat_hbm.at[P[2]], P[7], gsem).wait()
            pltpu.make_async_copy(sflat_hbm.at[P[3]], P[8], gsem).wait()
            pltpu.make_async_copy(dflat_hbm.at[P[4]], P[9], gsem).wait()
            pltpu.make_async_copy(dflat_hbm.at[P[5]], P[10], gsem).wait()

        def _fire_linear(ebase):
            pltpu.async_copy(src_hbm.at[pl.ds(ebase, 2 * K)], lsrc, lsem)
            pltpu.async_copy(dst_hbm.at[pl.ds(ebase, 2 * K)], ldst, lsem)

        def _wait_linear():
            pltpu.make_async_copy(src_hbm.at[pl.ds(0, 2 * K)], lsrc, lsem).wait()
            pltpu.make_async_copy(dst_hbm.at[pl.ds(0, 2 * K)], ldst, lsem).wait()

        def _compute(P, ebase):
            sv0, sv1, dv0, dv1, xhr, ee0, ee1, upd = P[7:]
            for g in range(K // 16):
                o = g * 16
                pos = ebase + o + iota16
                valid = pos < EACT
                for (svr, dvr, msp, eer) in ((sv0, dv0, msp0, ee0),
                                             (sv1, dv1, msp1, ee1)):
                    v = svr[pl.ds(o, 16)] + dvr[pl.ds(o, 16)]
                    v = jnp.where(v >= 0.0, v, v * LK)
                    v = jnp.exp(v - msp)
                    eer[pl.ds(o, 16)] = jnp.where(valid, v, 0.0)
            for g in range(K // 16):
                e0 = ee0[pl.ds(g * 16, 16)]
                e1 = ee1[pl.ds(g * 16, 16)]
                for r in range(16):
                    row = g * 16 + r
                    cidx = jnp.full((16,), r, jnp.int32)
                    sp0 = e0.at[cidx].get(mode="promise_in_bounds")
                    sp1 = e1.at[cidx].get(mode="promise_in_bounds")
                    upd[row, pl.ds(0, 16)] = xhr[row, pl.ds(0, 16)] * sp0
                    upd[row, pl.ds(16, 16)] = xhr[row, pl.ds(16, 16)] * sp1

        def _fire_scatters(P):
            pltpu.async_copy(P[12], den0_sh.at[P[6]], ssem, add=True)
            pltpu.async_copy(P[13], den1_sh.at[P[6]], ssem, add=True)
            pltpu.async_copy(P[14], out_sh.at[P[6]], ssem, add=True)

        def _drain_scatters(P):
            pltpu.make_async_copy(P[12], den0_sh.at[P[6]], ssem).wait()
            pltpu.make_async_copy(P[13], den1_sh.at[P[6]], ssem).wait()
            pltpu.make_async_copy(P[14], out_sh.at[P[6]], ssem).wait()

        NBODY = NCHUNK // 2
        _fire_linear(tilebase)

        def body(i2, carry):
            ea = pl.multiple_of(tilebase + i2 * (2 * K), K)
            eb = pl.multiple_of(ea + K, K)
            _wait_linear()

            @pl.when(i2 > 0)
            def _():
                _drain_scatters(P0)
                _drain_scatters(P1)

            _prep(P0, gsem0)
            _prep(P1, gsem1)
            enext = tilebase + (lax.rem(i2 + 1, NBODY)) * (2 * K)
            _fire_linear(pl.multiple_of(enext, K))
            _wait_gathers(P0, gsem0)
            _compute(P0, ea)
            _fire_scatters(P0)
            _wait_gathers(P1, gsem1)
            _compute(P1, eb)
            _fire_scatters(P1)
            return carry

        lax.fori_loop(0, NBODY, body, 0)
        _wait_linear()
        _drain_scatters(P0)
        _drain_scatters(P1)
        plsc.subcore_barrier()
        pltpu.sync_copy(out_sh.at[pl.ds(rowbase, RPT), :],
                        out_hbm.at[c, pl.ds(rowbase, RPT), :])

        @pl.when(c == 0)
        def _():
            pltpu.sync_copy(den0_sh.at[pl.ds(rowbase, RPT)],
                            den_a.at[pl.ds(rowbase, RPT)])
            pltpu.sync_copy(den1_sh.at[pl.ds(rowbase, RPT)],
                            den_b.at[pl.ds(rowbase, RPT)])

        @pl.when(c == 1)
        def _():
            pltpu.sync_copy(den0_sh.at[pl.ds(rowbase, RPT)],
                            den_c.at[pl.ds(rowbase, RPT)])
            pltpu.sync_copy(den1_sh.at[pl.ds(rowbase, RPT)],
                            den_d.at[pl.ds(rowbase, RPT)])


_SC_EDGE = pl.kernel(
    _sc_edge_kernel,
    out_type=(jax.ShapeDtypeStruct((NSC, NPAD, 32), jnp.float32),
              jax.ShapeDtypeStruct((NPAD,), jnp.float32),
              jax.ShapeDtypeStruct((NPAD,), jnp.float32),
              jax.ShapeDtypeStruct((NPAD,), jnp.float32),
              jax.ShapeDtypeStruct((NPAD,), jnp.float32)),
    mesh=plsc.VectorSubcoreMesh(core_axis_name="c", subcore_axis_name="s"),
    compiler_params=pltpu.CompilerParams(use_tc_tiling_on_sc=False),
    scratch_types=(
        [pltpu.VMEM((K,), jnp.int32),
         pltpu.VMEM((K,), jnp.int32),
         pltpu.VMEM((K,), jnp.int32),
         pltpu.VMEM((K,), jnp.int32),
         pltpu.VMEM((K,), jnp.int32),
         pltpu.VMEM((K,), jnp.int32),
         pltpu.VMEM((K,), jnp.float32),
         pltpu.VMEM((K,), jnp.float32),
         pltpu.VMEM((K,), jnp.float32),
         pltpu.VMEM((K,), jnp.float32),
         pltpu.VMEM((K, 32), jnp.float32),
         pltpu.VMEM((K,), jnp.float32),
         pltpu.VMEM((K,), jnp.float32),
         pltpu.VMEM((K, 32), jnp.float32),
         ] * 2
        + [pltpu.VMEM((2 * K,), jnp.int32),
           pltpu.VMEM((2 * K,), jnp.int32),
           pltpu.VMEM((16,), jnp.float32),
           pltpu.SemaphoreType.DMA,
           pltpu.SemaphoreType.DMA,
           pltpu.SemaphoreType.DMA,
           pltpu.SemaphoreType.DMA,
           pltpu.VMEM_SHARED((NPAD, 32), jnp.float32),
           pltpu.VMEM_SHARED((NPAD,), jnp.float32),
           pltpu.VMEM_SHARED((NPAD,), jnp.float32),
           ]),
)


def _tc_pre_specs():
    w = pl.BlockSpec((HID, HID), lambda i: (0, 0))
    v64 = pl.BlockSpec((1, HID), lambda i: (0, 0))
    s4 = pl.BlockSpec((HID, HEADS), lambda i: (0, 0))
    return w, v64, s4


def kernel(x, edge_index, batch, global_features, params):
    p = params
    f32 = jnp.float32

    ar = jnp.arange(N, dtype=jnp.int32)
    src = jnp.concatenate([edge_index[0].astype(jnp.int32), ar])
    dst = jnp.concatenate([edge_index[1].astype(jnp.int32), ar])
    src = jnp.pad(src, (0, EP - EACT))
    dst = jnp.pad(dst, (0, EP - EACT))

    x_p = jnp.pad(x, ((0, NPAD - N), (0, 0)))
    S = jnp.kron(jnp.eye(HEADS, dtype=f32), jnp.ones((CH, 1), f32))
    S2 = jnp.kron(jnp.eye(2, dtype=f32), jnp.ones((1, CH), f32))

    nblk = pl.BlockSpec((BLK, HID), lambda i: (i, 0))
    nblk4 = pl.BlockSpec((BLK, HEADS), lambda i: (i, 0))
    m4 = pl.BlockSpec((1, HEADS), lambda i: (0, 0))
    w, v64, s4 = _tc_pre_specs()

    h, xh, s, d, smax, dmax = pl.pallas_call(
        _enc_kernel,
        grid=(NB,),
        in_specs=[pl.BlockSpec((BLK, AF), lambda i: (i, 0)),
                  pl.BlockSpec((AF, HID), lambda i: (0, 0)), v64, w, v64,
                  w, v64, v64, s4],
        out_specs=[nblk, nblk, nblk4, nblk4, m4, m4],
        out_shape=[jax.ShapeDtypeStruct((NPAD, HID), f32),
                   jax.ShapeDtypeStruct((NPAD, HID), f32),
                   jax.ShapeDtypeStruct((NPAD, HEADS), f32),
                   jax.ShapeDtypeStruct((NPAD, HEADS), f32),
                   jax.ShapeDtypeStruct((1, HEADS), f32),
                   jax.ShapeDtypeStruct((1, HEADS), f32)],
    )(x_p, p['enc_W1'], p['enc_b1'].reshape(1, HID), p['enc_W2'],
      p['enc_b2'].reshape(1, HID), p['gat_W0'],
      p['att_src0'].reshape(1, HID), p['att_dst0'].reshape(1, HID), S)

    for l in range(NL):
        sflat = s.T.reshape(-1)
        dflat = d.T.reshape(-1)
        xh_cat = jnp.concatenate([xh[:, :32], xh[:, 32:]], axis=0)
        m16 = jnp.pad((smax + dmax).reshape(HEADS), (0, 12))
        out_sc, den_a, den_b, den_c, den_d = _SC_EDGE(
            src, dst, sflat, dflat, xh_cat, m16)
        den01 = jnp.stack([den_a, den_b], axis=-1).reshape(1, NPAD, 2)
        den23 = jnp.stack([den_c, den_d], axis=-1).reshape(1, NPAD, 2)

        common_in = [
            pl.BlockSpec((1, BLK, 32), lambda i: (0, i, 0)),
            pl.BlockSpec((1, BLK, 32), lambda i: (1, i, 0)),
            pl.BlockSpec((1, BLK, 2), lambda i: (0, i, 0)),
            pl.BlockSpec((1, BLK, 2), lambda i: (0, i, 0)),
            nblk, v64, v64, v64,
            pl.BlockSpec((2, 32), lambda i: (0, 0)),
        ]
        common_args = [out_sc, out_sc, den01, den23, h,
                       p[f'gat_b{l}'].reshape(1, HID),
                       p[f'ln_g{l}'].reshape(1, HID),
                       p[f'ln_b{l}'].reshape(1, HID), S2]
        if l < NL - 1:
            h, xh, s, d, smax, dmax = pl.pallas_call(
                _post_pre_kernel,
                grid=(NB,),
                in_specs=common_in + [w, v64, v64, s4],
                out_specs=[nblk, nblk, nblk4, nblk4, m4, m4],
                out_shape=[jax.ShapeDtypeStruct((NPAD, HID), f32),
                           jax.ShapeDtypeStruct((NPAD, HID), f32),
                           jax.ShapeDtypeStruct((NPAD, HEADS), f32),
                           jax.ShapeDtypeStruct((NPAD, HEADS), f32),
                           jax.ShapeDtypeStruct((1, HEADS), f32),
                           jax.ShapeDtypeStruct((1, HEADS), f32)],
            )(*common_args, p[f'gat_W{l + 1}'],
              p[f'att_src{l + 1}'].reshape(1, HID),
              p[f'att_dst{l + 1}'].reshape(1, HID), S)
        else:
            batch_p = jnp.pad(batch.astype(jnp.int32), (0, NPAD - N),
                              constant_values=B).astype(f32).reshape(NPAD, 1)
            hsum, cnt = pl.pallas_call(
                _post_pool_kernel,
                grid=(NB,),
                in_specs=common_in + [pl.BlockSpec((BLK, 1), lambda i: (i, 0))],
                out_specs=[pl.BlockSpec((B, HID), lambda i: (0, 0)),
                           pl.BlockSpec((1, B), lambda i: (0, 0))],
                out_shape=[jax.ShapeDtypeStruct((B, HID), f32),
                           jax.ShapeDtypeStruct((1, B), f32)],
            )(*common_args, batch_p)

    logs = jnp.stack([p['log_lambda'], p['log_alpha'],
                      p['log_omega_max']]).reshape(1, 3)
    cv, cv_physics, cv_direct, omega_eff, omega_raw, causality = pl.pallas_call(
        _heads_kernel,
        out_shape=[jax.ShapeDtypeStruct((B, 1), f32)] * 6,
    )(hsum, cnt.reshape(B, 1), global_features, p['glob_W'],
      p['glob_b'].reshape(1, HID), p['om_W1'], p['om_b1'].reshape(1, HID),
      p['om_W2'], p['om_b2'].reshape(1, 1), p['cv_W1'],
      p['cv_b1'].reshape(1, HID), p['cv_W2'], p['cv_b2'].reshape(1, 1), logs)

    lmbda = jnp.exp(p['log_lambda'])
    alpha = jnp.exp(p['log_alpha'])
    omega_max = jnp.exp(p['log_omega_max'])
    return (cv, cv_physics, cv_direct, omega_eff, omega_raw, causality,
            lmbda, alpha, omega_max)


# speedup vs baseline: 119.5735x; 1.0664x over previous
import functools

import jax
import jax.numpy as jnp
from jax import lax
from jax.experimental import pallas as pl
from jax.experimental.pallas import tpu as pltpu
from jax.experimental.pallas import tpu_sc as plsc

N = 50000
E = 800000
B = 128
AF = 21
HID = 64
HEADS = 4
CH = 16
NL = 4

BLK = 1024
NPAD = 50176
NB = NPAD // BLK
EACT = E + N
NSC = 2
NTI = 16
K = 128
NCHUNK = 416
TPT = K * NCHUNK
EP = TPT * NTI
RPT = NPAD // NTI
LK = 0.2
XW = 34


def _dense_pre(h, W, asrc, adst, S):
    xh = jnp.dot(h, W, preferred_element_type=jnp.float32)
    s = jnp.dot(xh * asrc, S, preferred_element_type=jnp.float32)
    d = jnp.dot(xh * adst, S, preferred_element_type=jnp.float32)
    return xh[:, :32], xh[:, 32:], s, d


def _acc_max(ref, blk, i):
    bm = jnp.max(blk, axis=0, keepdims=True)

    @pl.when(i == 0)
    def _():
        ref[...] = bm

    @pl.when(i > 0)
    def _():
        ref[...] = jnp.maximum(ref[...], bm)


def _enc_kernel(x_ref, w1_ref, b1_ref, w2_ref, b2_ref, W_ref, as_ref,
                ad_ref, S_ref, h_ref, xa_ref, xb_ref, s_ref, d_ref, sm_ref,
                dm_ref):
    i = pl.program_id(0)
    x = x_ref[...]
    t = jnp.maximum(jnp.dot(x, w1_ref[...], preferred_element_type=jnp.float32)
                    + b1_ref[...], 0.0)
    h = jnp.dot(t, w2_ref[...], preferred_element_type=jnp.float32) + b2_ref[...]
    h_ref[...] = h
    xa, xb, s, d = _dense_pre(h, W_ref[...], as_ref[...], ad_ref[...],
                              S_ref[...])
    xa_ref[...] = xa
    xb_ref[...] = xb
    s_ref[...] = s
    d_ref[...] = d
    _acc_max(sm_ref, s, i)
    _acc_max(dm_ref, d, i)


def _post_body(o0_ref, o1_ref, dn0_ref, dn1_ref, hp_ref, bias_ref, lg_ref,
               lb_ref, S2_ref):
    d0 = jnp.dot(dn0_ref[0], S2_ref[...], preferred_element_type=jnp.float32)
    d1 = jnp.dot(dn1_ref[0], S2_ref[...], preferred_element_type=jnp.float32)
    g0 = o0_ref[0] / (d0 + 1e-16)
    g1 = o1_ref[0] / (d1 + 1e-16)
    g = jnp.concatenate([g0, g1], axis=1) + bias_ref[...]
    t = g + hp_ref[...]
    m = jnp.mean(t, axis=-1, keepdims=True)
    v = jnp.mean((t - m) * (t - m), axis=-1, keepdims=True)
    t = (t - m) / jnp.sqrt(v + 1e-5) * lg_ref[...] + lb_ref[...]
    return jnp.maximum(t, 0.0)


def _post_pre_kernel(o0_ref, o1_ref, dn0_ref, dn1_ref, hp_ref, bias_ref,
                     lg_ref, lb_ref, S2_ref, W_ref, as_ref, ad_ref, S_ref,
                     h_ref, xa_ref, xb_ref, s_ref, d_ref, sm_ref, dm_ref):
    i = pl.program_id(0)
    h = _post_body(o0_ref, o1_ref, dn0_ref, dn1_ref, hp_ref, bias_ref, lg_ref,
                   lb_ref, S2_ref)
    h_ref[...] = h
    xa, xb, s, d = _dense_pre(h, W_ref[...], as_ref[...], ad_ref[...],
                              S_ref[...])
    xa_ref[...] = xa
    xb_ref[...] = xb
    s_ref[...] = s
    d_ref[...] = d
    _acc_max(sm_ref, s, i)
    _acc_max(dm_ref, d, i)


def _post_pool_kernel(o0_ref, o1_ref, dn0_ref, dn1_ref, hp_ref, bias_ref,
                      lg_ref, lb_ref, S2_ref, batch_ref, hsum_ref, cnt_ref):
    i = pl.program_id(0)
    h = _post_body(o0_ref, o1_ref, dn0_ref, dn1_ref, hp_ref, bias_ref, lg_ref,
                   lb_ref, S2_ref)
    iota_b = lax.broadcasted_iota(jnp.int32, (1, B), 1).astype(jnp.float32)
    onehot = (batch_ref[...] == iota_b).astype(jnp.float32)
    hs = lax.dot_general(onehot, h, (((0,), (0,)), ((), ())),
                         preferred_element_type=jnp.float32)
    cn = jnp.sum(onehot, axis=0, keepdims=True)

    @pl.when(i == 0)
    def _():
        hsum_ref[...] = hs
        cnt_ref[...] = cn

    @pl.when(i > 0)
    def _():
        hsum_ref[...] = hsum_ref[...] + hs
        cnt_ref[...] = cnt_ref[...] + cn


def _heads_kernel(hsum_ref, cnt_ref, gf_ref, gw_ref, gb_ref,
                  ow1_ref, ob1_ref, ow2_ref, ob2_ref,
                  cw1_ref, cb1_ref, cw2_ref, cb2_ref, logs_ref,
                  cv_ref, cvp_ref, cvd_ref, oe_ref, or_ref, ca_ref):
    cnt = jnp.maximum(cnt_ref[...], 1.0)
    h_graph = hsum_ref[...] / cnt
    h_glob = jnp.dot(gf_ref[...], gw_ref[...],
                     preferred_element_type=jnp.float32) + gb_ref[...]
    hc = jnp.concatenate([h_graph, h_glob], axis=1)
    th = jnp.maximum(jnp.dot(hc, ow1_ref[...],
                             preferred_element_type=jnp.float32) + ob1_ref[...], 0.0)
    omega_raw = jax.nn.softplus(
        jnp.dot(th, ow2_ref[...], preferred_element_type=jnp.float32) + ob2_ref[...])
    tc = jnp.maximum(jnp.dot(hc, cw1_ref[...],
                             preferred_element_type=jnp.float32) + cb1_ref[...], 0.0)
    cv_direct = jax.nn.softplus(
        jnp.dot(tc, cw2_ref[...], preferred_element_type=jnp.float32) + cb2_ref[...])
    lmbda = jnp.exp(logs_ref[0, 0])
    alpha = jnp.exp(logs_ref[0, 1])
    omega_max = jnp.exp(logs_ref[0, 2])
    omega_eff = jnp.maximum(jnp.minimum(alpha * omega_raw, omega_max), 2.0)
    causality = jnp.exp(-lmbda * jnp.log(omega_eff))
    cv_physics = 30.0 * (1.0 - causality)
    cv_ref[...] = 0.5 * cv_physics + 0.5 * cv_direct
    cvp_ref[...] = cv_physics
    cvd_ref[...] = cv_direct
    oe_ref[...] = omega_eff
    or_ref[...] = omega_raw
    ca_ref[...] = causality


def _sc_edge_kernel(src_hbm, dst_hbm, sflat_hbm, dflat_hbm, xa_hbm, xb_hbm,
                    m_hbm, out_hbm, den_a, den_b, den_c, den_d,
                    sxi0, s0i0, s1i0, d0i0, d1i0, dsti0,
                    sv00, sv10, dv00, dv10, xhr0, ee00, ee10, upd0,
                    sxi1, s0i1, s1i1, d0i1, d1i1, dsti1,
                    sv01, sv11, dv01, dv11, xhr1, ee01, ee11, upd1,
                    lsrc, ldst, m16v, gsem0, gsem1, ssem, lsem,
                    out_sh, den0_sh, den1_sh):
    c = lax.axis_index("c")
    t_id = lax.axis_index("s")
    rowbase = t_id * RPT
    tilebase = t_id * TPT
    P0 = (0, sxi0, s0i0, s1i0, d0i0, d1i0, dsti0,
          sv00, sv10, dv00, dv10, xhr0, ee00, ee10, upd0)
    P1 = (K, sxi1, s0i1, s1i1, d0i1, d1i1, dsti1,
          sv01, sv11, dv01, dv11, xhr1, ee01, ee11, upd1)

    def _zero(out_sh, den0_sh, den1_sh):
        zv = jnp.zeros((16,), jnp.float32)
        for r in range(K):
            upd0[r, pl.ds(0, 16)] = zv
            upd0[r, pl.ds(16, 16)] = zv
        for g in range(K // 16):
            ee00[pl.ds(g * 16, 16)] = zv
        nfull = RPT // K
        for j in range(nfull):
            pltpu.sync_copy(upd0, out_sh.at[pl.ds(rowbase + j * K, K), :])
            pltpu.sync_copy(ee00, den0_sh.at[pl.ds(rowbase + j * K, K)])
            pltpu.sync_copy(ee00, den1_sh.at[pl.ds(rowbase + j * K, K)])
        r = RPT - nfull * K
        pltpu.sync_copy(upd0.at[pl.ds(0, r), :],
                        out_sh.at[pl.ds(rowbase + nfull * K, r), :])
        pltpu.sync_copy(ee00.at[pl.ds(0, r)],
                        den0_sh.at[pl.ds(rowbase + nfull * K, r)])
        pltpu.sync_copy(ee00.at[pl.ds(0, r)],
                        den1_sh.at[pl.ds(rowbase + nfull * K, r)])

    if True:
        pltpu.sync_copy(m_hbm, m16v)
        _zero(out_sh, den0_sh, den1_sh)
        plsc.subcore_barrier()

        mvec = m16v[...]
        iota16 = lax.broadcasted_iota(jnp.int32, (16,), 0)
        h0 = 2 * c
        msp0 = mvec.at[jnp.broadcast_to(h0, (16,))].get(mode="promise_in_bounds")
        msp1 = mvec.at[jnp.broadcast_to(h0 + 1, (16,))].get(mode="promise_in_bounds")
        offs0 = h0 * NPAD
        offs1 = (h0 + 1) * NPAD

        def _prep(P, gsem):
            off, sxi, s0i, s1i, d0i, d1i, dsti = P[:7]
            for g in range(K // 16):
                o = g * 16
                sv = lsrc[pl.ds(off + o, 16)]
                dv = ldst[pl.ds(off + o, 16)]
                sxi[pl.ds(o, 16)] = sv
                s0i[pl.ds(o, 16)] = sv + offs0
                s1i[pl.ds(o, 16)] = sv + offs1
                d0i[pl.ds(o, 16)] = dv + offs0
                d1i[pl.ds(o, 16)] = dv + offs1
                dsti[pl.ds(o, 16)] = dv

            @pl.when(c == 0)
            def _():
                pltpu.async_copy(xa_hbm.at[P[1]], P[11], gsem)

            @pl.when(c == 1)
            def _():
                pltpu.async_copy(xb_hbm.at[P[1]], P[11], gsem)

            pltpu.async_copy(sflat_hbm.at[P[2]], P[7], gsem)
            pltpu.async_copy(sflat_hbm.at[P[3]], P[8], gsem)
            pltpu.async_copy(dflat_hbm.at[P[4]], P[9], gsem)
            pltpu.async_copy(dflat_hbm.at[P[5]], P[10], gsem)

        def _wait_gathers(P, gsem):
            pltpu.make_async_copy(xa_hbm.at[P[1]], P[11], gsem).wait()
            pltpu.make_async_copy(sflat_hbm.at[P[2]], P[7], gsem).wait()
            pltpu.make_async_copy(sflat_hbm.at[P[3]], P[8], gsem).wait()
            pltpu.make_async_copy(dflat_hbm.at[P[4]], P[9], gsem).wait()
            pltpu.make_async_copy(dflat_hbm.at[P[5]], P[10], gsem).wait()

        def _fire_linear(ebase):
            pltpu.async_copy(src_hbm.at[pl.ds(ebase, 2 * K)], lsrc, lsem)
            pltpu.async_copy(dst_hbm.at[pl.ds(ebase, 2 * K)], ldst, lsem)

        def _wait_linear():
            pltpu.make_async_copy(src_hbm.at[pl.ds(0, 2 * K)], lsrc, lsem).wait()
            pltpu.make_async_copy(dst_hbm.at[pl.ds(0, 2 * K)], ldst, lsem).wait()

        def _compute(P, ebase):
            sv0, sv1, dv0, dv1, xhr, ee0, ee1, upd = P[7:]
            for g in range(K // 16):
                o = g * 16
                pos = ebase + o + iota16
                valid = pos < EACT
                for (svr, dvr, msp, eer) in ((sv0, dv0, msp0, ee0),
                                             (sv1, dv1, msp1, ee1)):
                    v = svr[pl.ds(o, 16)] + dvr[pl.ds(o, 16)]
                    v = jnp.where(v >= 0.0, v, v * LK)
                    v = jnp.exp(v - msp)
                    eer[pl.ds(o, 16)] = jnp.where(valid, v, 0.0)
            for g in range(K // 16):
                e0 = ee0[pl.ds(g * 16, 16)]
                e1 = ee1[pl.ds(g * 16, 16)]
                for r in range(16):
                    row = g * 16 + r
                    cidx = jnp.full((16,), r, jnp.int32)
                    sp0 = e0.at[cidx].get(mode="promise_in_bounds")
                    sp1 = e1.at[cidx].get(mode="promise_in_bounds")
                    upd[row, pl.ds(0, 16)] = xhr[row, pl.ds(0, 16)] * sp0
                    upd[row, pl.ds(16, 16)] = xhr[row, pl.ds(16, 16)] * sp1

        def _fire_scatters(P):
            pltpu.async_copy(P[12], den0_sh.at[P[6]], ssem, add=True)
            pltpu.async_copy(P[13], den1_sh.at[P[6]], ssem, add=True)
            pltpu.async_copy(P[14], out_sh.at[P[6]], ssem, add=True)

        def _drain_scatters(P):
            pltpu.make_async_copy(P[12], den0_sh.at[P[6]], ssem).wait()
            pltpu.make_async_copy(P[13], den1_sh.at[P[6]], ssem).wait()
            pltpu.make_async_copy(P[14], out_sh.at[P[6]], ssem).wait()

        NBODY = NCHUNK // 2
        _fire_linear(tilebase)

        def body(i2, carry):
            ea = pl.multiple_of(tilebase + i2 * (2 * K), K)
            eb = pl.multiple_of(ea + K, K)
            _wait_linear()

            @pl.when(i2 > 0)
            def _():
                _drain_scatters(P0)
                _drain_scatters(P1)

            _prep(P0, gsem0)
            _prep(P1, gsem1)
            enext = tilebase + (lax.rem(i2 + 1, NBODY)) * (2 * K)
            _fire_linear(pl.multiple_of(enext, K))
            _wait_gathers(P0, gsem0)
            _compute(P0, ea)
            _fire_scatters(P0)
            _wait_gathers(P1, gsem1)
            _compute(P1, eb)
            _fire_scatters(P1)
            return carry

        lax.fori_loop(0, NBODY, body, 0)
        _wait_linear()
        _drain_scatters(P0)
        _drain_scatters(P1)
        plsc.subcore_barrier()
        pltpu.sync_copy(out_sh.at[pl.ds(rowbase, RPT), :],
                        out_hbm.at[c, pl.ds(rowbase, RPT), :])

        @pl.when(c == 0)
        def _():
            pltpu.sync_copy(den0_sh.at[pl.ds(rowbase, RPT)],
                            den_a.at[pl.ds(rowbase, RPT)])
            pltpu.sync_copy(den1_sh.at[pl.ds(rowbase, RPT)],
                            den_b.at[pl.ds(rowbase, RPT)])

        @pl.when(c == 1)
        def _():
            pltpu.sync_copy(den0_sh.at[pl.ds(rowbase, RPT)],
                            den_c.at[pl.ds(rowbase, RPT)])
            pltpu.sync_copy(den1_sh.at[pl.ds(rowbase, RPT)],
                            den_d.at[pl.ds(rowbase, RPT)])


_SC_EDGE = pl.kernel(
    _sc_edge_kernel,
    out_type=(jax.ShapeDtypeStruct((NSC, NPAD, 32), jnp.float32),
              jax.ShapeDtypeStruct((NPAD,), jnp.float32),
              jax.ShapeDtypeStruct((NPAD,), jnp.float32),
              jax.ShapeDtypeStruct((NPAD,), jnp.float32),
              jax.ShapeDtypeStruct((NPAD,), jnp.float32)),
    mesh=plsc.VectorSubcoreMesh(core_axis_name="c", subcore_axis_name="s"),
    compiler_params=pltpu.CompilerParams(use_tc_tiling_on_sc=False),
    scratch_types=(
        [pltpu.VMEM((K,), jnp.int32),
         pltpu.VMEM((K,), jnp.int32),
         pltpu.VMEM((K,), jnp.int32),
         pltpu.VMEM((K,), jnp.int32),
         pltpu.VMEM((K,), jnp.int32),
         pltpu.VMEM((K,), jnp.int32),
         pltpu.VMEM((K,), jnp.float32),
         pltpu.VMEM((K,), jnp.float32),
         pltpu.VMEM((K,), jnp.float32),
         pltpu.VMEM((K,), jnp.float32),
         pltpu.VMEM((K, 32), jnp.float32),
         pltpu.VMEM((K,), jnp.float32),
         pltpu.VMEM((K,), jnp.float32),
         pltpu.VMEM((K, 32), jnp.float32),
         ] * 2
        + [pltpu.VMEM((2 * K,), jnp.int32),
           pltpu.VMEM((2 * K,), jnp.int32),
           pltpu.VMEM((16,), jnp.float32),
           pltpu.SemaphoreType.DMA,
           pltpu.SemaphoreType.DMA,
           pltpu.SemaphoreType.DMA,
           pltpu.SemaphoreType.DMA,
           pltpu.VMEM_SHARED((NPAD, 32), jnp.float32),
           pltpu.VMEM_SHARED((NPAD,), jnp.float32),
           pltpu.VMEM_SHARED((NPAD,), jnp.float32),
           ]),
)


def _tc_pre_specs():
    w = pl.BlockSpec((HID, HID), lambda i: (0, 0))
    v64 = pl.BlockSpec((1, HID), lambda i: (0, 0))
    s4 = pl.BlockSpec((HID, HEADS), lambda i: (0, 0))
    return w, v64, s4


def kernel(x, edge_index, batch, global_features, params):
    p = params
    f32 = jnp.float32

    ar = jnp.arange(N, dtype=jnp.int32)
    src = jnp.concatenate([edge_index[0].astype(jnp.int32), ar])
    dst = jnp.concatenate([edge_index[1].astype(jnp.int32), ar])
    src = jnp.pad(src, (0, EP - EACT))
    dst = jnp.pad(dst, (0, EP - EACT))

    x_p = jnp.pad(x, ((0, NPAD - N), (0, 0)))
    S = jnp.kron(jnp.eye(HEADS, dtype=f32), jnp.ones((CH, 1), f32))
    S2 = jnp.kron(jnp.eye(2, dtype=f32), jnp.ones((1, CH), f32))

    nblk = pl.BlockSpec((BLK, HID), lambda i: (i, 0))
    nblk4 = pl.BlockSpec((BLK, HEADS), lambda i: (i, 0))
    m4 = pl.BlockSpec((1, HEADS), lambda i: (0, 0))
    w, v64, s4 = _tc_pre_specs()

    nblk32 = pl.BlockSpec((BLK, 32), lambda i: (i, 0))
    pre_outs = dict(
        out_specs=[nblk, nblk32, nblk32, nblk4, nblk4, m4, m4],
        out_shape=[jax.ShapeDtypeStruct((NPAD, HID), f32),
                   jax.ShapeDtypeStruct((NPAD, 32), f32),
                   jax.ShapeDtypeStruct((NPAD, 32), f32),
                   jax.ShapeDtypeStruct((NPAD, HEADS), f32),
                   jax.ShapeDtypeStruct((NPAD, HEADS), f32),
                   jax.ShapeDtypeStruct((1, HEADS), f32),
                   jax.ShapeDtypeStruct((1, HEADS), f32)])

    h, xa, xb, s, d, smax, dmax = pl.pallas_call(
        _enc_kernel,
        grid=(NB,),
        in_specs=[pl.BlockSpec((BLK, AF), lambda i: (i, 0)),
                  pl.BlockSpec((AF, HID), lambda i: (0, 0)), v64, w, v64,
                  w, v64, v64, s4],
        **pre_outs,
    )(x_p, p['enc_W1'], p['enc_b1'].reshape(1, HID), p['enc_W2'],
      p['enc_b2'].reshape(1, HID), p['gat_W0'],
      p['att_src0'].reshape(1, HID), p['att_dst0'].reshape(1, HID), S)

    for l in range(NL):
        sflat = s.T.reshape(-1)
        dflat = d.T.reshape(-1)
        m16 = jnp.pad((smax + dmax).reshape(HEADS), (0, 12))
        out_sc, den_a, den_b, den_c, den_d = _SC_EDGE(
            src, dst, sflat, dflat, xa, xb, m16)
        den01 = jnp.stack([den_a, den_b], axis=-1).reshape(1, NPAD, 2)
        den23 = jnp.stack([den_c, den_d], axis=-1).reshape(1, NPAD, 2)

        common_in = [
            pl.BlockSpec((1, BLK, 32), lambda i: (0, i, 0)),
            pl.BlockSpec((1, BLK, 32), lambda i: (1, i, 0)),
            pl.BlockSpec((1, BLK, 2), lambda i: (0, i, 0)),
            pl.BlockSpec((1, BLK, 2), lambda i: (0, i, 0)),
            nblk, v64, v64, v64,
            pl.BlockSpec((2, 32), lambda i: (0, 0)),
        ]
        common_args = [out_sc, out_sc, den01, den23, h,
                       p[f'gat_b{l}'].reshape(1, HID),
                       p[f'ln_g{l}'].reshape(1, HID),
                       p[f'ln_b{l}'].reshape(1, HID), S2]
        if l < NL - 1:
            h, xa, xb, s, d, smax, dmax = pl.pallas_call(
                _post_pre_kernel,
                grid=(NB,),
                in_specs=common_in + [w, v64, v64, s4],
                **pre_outs,
            )(*common_args, p[f'gat_W{l + 1}'],
              p[f'att_src{l + 1}'].reshape(1, HID),
              p[f'att_dst{l + 1}'].reshape(1, HID), S)
        else:
            batch_p = jnp.pad(batch.astype(jnp.int32), (0, NPAD - N),
                              constant_values=B).astype(f32).reshape(NPAD, 1)
            hsum, cnt = pl.pallas_call(
                _post_pool_kernel,
                grid=(NB,),
                in_specs=common_in + [pl.BlockSpec((BLK, 1), lambda i: (i, 0))],
                out_specs=[pl.BlockSpec((B, HID), lambda i: (0, 0)),
                           pl.BlockSpec((1, B), lambda i: (0, 0))],
                out_shape=[jax.ShapeDtypeStruct((B, HID), f32),
                           jax.ShapeDtypeStruct((1, B), f32)],
            )(*common_args, batch_p)

    logs = jnp.stack([p['log_lambda'], p['log_alpha'],
                      p['log_omega_max']]).reshape(1, 3)
    cv, cv_physics, cv_direct, omega_eff, omega_raw, causality = pl.pallas_call(
        _heads_kernel,
        out_shape=[jax.ShapeDtypeStruct((B, 1), f32)] * 6,
    )(hsum, cnt.reshape(B, 1), global_features, p['glob_W'],
      p['glob_b'].reshape(1, HID), p['om_W1'], p['om_b1'].reshape(1, HID),
      p['om_W2'], p['om_b2'].reshape(1, 1), p['cv_W1'],
      p['cv_b1'].reshape(1, HID), p['cv_W2'], p['cv_b2'].reshape(1, 1), logs)

    lmbda = jnp.exp(p['log_lambda'])
    alpha = jnp.exp(p['log_alpha'])
    omega_max = jnp.exp(p['log_omega_max'])
    return (cv, cv_physics, cv_direct, omega_eff, omega_raw, causality,
            lmbda, alpha, omega_max)
